# Initial kernel scaffold; baseline (speedup 1.0000x reference)
import functools

import jax
import jax.numpy as jnp
from jax import lax
from jax.experimental import pallas as pl
from jax.experimental.pallas import tpu as pltpu
from jax.experimental.pallas import tpu_sc as plsc

N = 10000
E = 320000
CH = 128
HALF = 64
NC, NS, L = 2, 16, 16
B = 128
E_PAD = ((E + NS * B - 1) // (NS * B)) * (NS * B)
T = E_PAD // NS
NBATCH = T // B
ROWS_PER_TILE = (2 * N) // NS
DEG_CHUNK = 2000


def _sc_aggregate(x2, src, dst, attr):
    mesh = plsc.VectorSubcoreMesh(
        core_axis_name="c", subcore_axis_name="s", num_cores=NC, num_subcores=NS
    )

    @functools.partial(
        pl.kernel,
        out_type=[
            jax.ShapeDtypeStruct((NC, 2 * N, HALF), jnp.float32),
            jax.ShapeDtypeStruct((NC, 2 * N), jnp.float32),
        ],
        mesh=mesh,
        scratch_types=[
            pltpu.VMEM_SHARED((2 * N, HALF), jnp.float32),
            pltpu.VMEM_SHARED((2 * N,), jnp.float32),
            pltpu.VMEM((T,), jnp.int32),
            pltpu.VMEM((T,), jnp.int32),
            pltpu.VMEM((T,), jnp.float32),
            pltpu.VMEM((B,), jnp.int32),
            pltpu.VMEM((B,), jnp.int32),
            pltpu.VMEM((B,), jnp.float32),
            pltpu.VMEM((B, HALF), jnp.float32),
            pltpu.VMEM((DEG_CHUNK,), jnp.float32),
            pltpu.SemaphoreType.DMA,
        ],
    )
    def sc_kernel(x2_hbm, src_hbm, dst_hbm, attr_hbm, acc_out, deg_out,
                  acc_sh, deg_sh, src_v, dst_v, attr_v, ridx_v, sidx_v,
                  w_v, rows_v, zd_v, sem):
        cid = lax.axis_index("c")
        sid = lax.axis_index("s")

        zero16 = jnp.zeros((L,), jnp.float32)
        for r in range(B):
            for j in range(HALF // L):
                rows_v[r, pl.ds(j * L, L)] = zero16
        for j in range(DEG_CHUNK // L):
            zd_v[pl.ds(j * L, L)] = zero16
        r0 = sid * ROWS_PER_TILE
        for k in range(9):
            pltpu.sync_copy(rows_v, acc_sh.at[pl.ds(r0 + k * B, B)])
        rem = ROWS_PER_TILE - 9 * B
        pltpu.sync_copy(rows_v.at[pl.ds(0, rem)],
                        acc_sh.at[pl.ds(r0 + 9 * B, rem)])

        @pl.when(sid < 10)
        def _zero_deg():
            pltpu.sync_copy(zd_v, deg_sh.at[pl.ds(sid * DEG_CHUNK, DEG_CHUNK)])

        plsc.subcore_barrier()

        tbase = sid * T
        pltpu.sync_copy(src_hbm.at[pl.ds(tbase, T)], src_v)
        pltpu.sync_copy(dst_hbm.at[pl.ds(tbase, T)], dst_v)
        pltpu.sync_copy(attr_hbm.at[pl.ds(tbase, T)], attr_v)

        def batch_body(b, carry):
            off = b * B
            for g in range(B // L):
                sl = pl.ds(off + g * L, L)
                s16 = src_v[sl]
                d16 = dst_v[sl]
                a16 = attr_v[sl]
                gl = pl.ds(g * L, L)
                ridx_v[gl] = s16 + cid * N
                sidx_v[gl] = d16 + jnp.where(a16 < 0.0, N, 0)
                w_v[gl] = jnp.abs(a16)
            pltpu.async_copy(x2_hbm.at[ridx_v], rows_v, sem).wait()
            for i in range(B):
                wv = jnp.full((L,), w_v[i], jnp.float32)
                for j in range(HALF // L):
                    cs = pl.ds(j * L, L)
                    rows_v[i, cs] = rows_v[i, cs] * wv
            pltpu.sync_copy(rows_v, acc_sh.at[sidx_v], add=True)
            pltpu.sync_copy(w_v, deg_sh.at[sidx_v], add=True)
            return carry

        lax.fori_loop(0, NBATCH, batch_body, 0)
        plsc.subcore_barrier()

        pltpu.sync_copy(acc_sh.at[pl.ds(r0, ROWS_PER_TILE)],
                        acc_out.at[cid, pl.ds(r0, ROWS_PER_TILE)])

        @pl.when(sid < 10)
        def _write_deg():
            d0 = sid * DEG_CHUNK
            pltpu.sync_copy(deg_sh.at[pl.ds(d0, DEG_CHUNK)],
                            deg_out.at[cid, pl.ds(d0, DEG_CHUNK)])

    return sc_kernel(x2, src, dst, attr)


def _tc_dense(acc, deg, x, W_pos_l, W_pos_r, b_pos, W_neg_l, W_neg_r, b_neg):
    R = 500
    grid = (N // R,)

    def body(a_ref, deg_ref, x_ref, wpl, wpr, bp, wnl, wnr, bn, o_ref):
        a = a_ref[...]
        pos = jnp.concatenate([a[0, 0], a[1, 0]], axis=-1)
        neg = jnp.concatenate([a[0, 1], a[1, 1]], axis=-1)
        dg = deg_ref[...]
        dp = jnp.where(dg[0] > 0.0, dg[0], 1.0)
        dn = jnp.where(dg[1] > 0.0, dg[1], 1.0)
        pos = pos / dp[:, None]
        neg = neg / dn[:, None]
        xb = x_ref[...]
        dims = (((1,), (1,)), ((), ()))
        op = (lax.dot_general(pos, wpl[...], dims, preferred_element_type=jnp.float32)
              + lax.dot_general(xb, wpr[...], dims, preferred_element_type=jnp.float32)
              + bp[...])
        on = (lax.dot_general(neg, wnl[...], dims, preferred_element_type=jnp.float32)
              + lax.dot_general(xb, wnr[...], dims, preferred_element_type=jnp.float32)
              + bn[...])
        o_ref[...] = jnp.maximum(jnp.concatenate([op, on], axis=-1), 0.0)

    return pl.pallas_call(
        body,
        grid=grid,
        in_specs=[
            pl.BlockSpec((2, 2, R, HALF), lambda i: (0, 0, i, 0)),
            pl.BlockSpec((2, R), lambda i: (0, i)),
            pl.BlockSpec((R, CH), lambda i: (i, 0)),
            pl.BlockSpec((CH, CH), lambda i: (0, 0)),
            pl.BlockSpec((CH, CH), lambda i: (0, 0)),
            pl.BlockSpec((1, CH), lambda i: (0, 0)),
            pl.BlockSpec((CH, CH), lambda i: (0, 0)),
            pl.BlockSpec((CH, CH), lambda i: (0, 0)),
            pl.BlockSpec((1, CH), lambda i: (0, 0)),
        ],
        out_specs=pl.BlockSpec((R, 2 * CH), lambda i: (i, 0)),
        out_shape=jax.ShapeDtypeStruct((N, 2 * CH), jnp.float32),
    )(acc, deg, x, W_pos_l, W_pos_r, b_pos.reshape(1, CH),
      W_neg_l, W_neg_r, b_neg.reshape(1, CH))


def kernel(x, edge_index, edge_attr, W_pos_l, W_pos_r, b_pos,
           W_neg_l, W_neg_r, b_neg):
    src = edge_index[0].astype(jnp.int32)
    dst = edge_index[1].astype(jnp.int32)
    pad = E_PAD - E
    src = jnp.pad(src, (0, pad))
    dst = jnp.pad(dst, (0, pad))
    attr = jnp.pad(edge_attr, (0, pad))
    x2 = jnp.concatenate([x[:, :HALF], x[:, HALF:]], axis=0)

    acc, deg = _sc_aggregate(x2, src, dst, attr)
    acc = acc.reshape(NC, 2, N, HALF)
    deg = deg[0].reshape(2, N)
    return _tc_dense(acc, deg, x, W_pos_l, W_pos_r, b_pos,
                     W_neg_l, W_neg_r, b_neg)


# speedup vs baseline: 5.3607x; 5.3607x over previous
import functools

import jax
import jax.numpy as jnp
from jax import lax
from jax.experimental import pallas as pl
from jax.experimental.pallas import tpu as pltpu
from jax.experimental.pallas import tpu_sc as plsc

N = 10000
E = 320000
CH = 128
HALF = 64
NC, NS, L = 2, 16, 16
B = 128
CK = 1024
E_PAD = ((E + NS * CK - 1) // (NS * CK)) * (NS * CK)
T = E_PAD // NS
NCHUNK = T // CK
ROWS_PER_TILE = (2 * N) // NS
DEG_CHUNK = 2000


def _sc_aggregate(x2, src, dst, attr):
    mesh = plsc.VectorSubcoreMesh(
        core_axis_name="c", subcore_axis_name="s", num_cores=NC, num_subcores=NS
    )

    @functools.partial(
        pl.kernel,
        out_type=[
            jax.ShapeDtypeStruct((NC, NS, ROWS_PER_TILE, HALF), jnp.float32),
            jax.ShapeDtypeStruct((NC, 10, DEG_CHUNK), jnp.float32),
        ],
        mesh=mesh,
        compiler_params=pltpu.CompilerParams(
            needs_layout_passes=False, use_tc_tiling_on_sc=False),
        scratch_types=[
            pltpu.VMEM_SHARED((2 * N, HALF), jnp.float32),
            pltpu.VMEM_SHARED((2 * N,), jnp.float32),
            pltpu.VMEM((CK,), jnp.int32),
            pltpu.VMEM((CK,), jnp.int32),
            pltpu.VMEM((CK,), jnp.float32),
            pltpu.VMEM((B,), jnp.int32),
            pltpu.VMEM((B,), jnp.int32),
            pltpu.VMEM((B,), jnp.float32),
            pltpu.VMEM((B, HALF), jnp.float32),
            pltpu.VMEM((DEG_CHUNK,), jnp.float32),
            pltpu.SemaphoreType.DMA,
        ],
    )
    def sc_kernel(x2_hbm, src_hbm, dst_hbm, attr_hbm, acc_out, deg_out,
                  acc_sh, deg_sh, src_v, dst_v, attr_v, ridx_v, sidx_v,
                  w_v, rows_v, zd_v, sem):
        cid = lax.axis_index("c")
        sid = lax.axis_index("s")

        zero16 = jnp.zeros((L,), jnp.float32)
        for r in range(B):
            for j in range(HALF // L):
                rows_v[r, pl.ds(j * L, L)] = zero16
        for j in range(DEG_CHUNK // L):
            zd_v[pl.ds(j * L, L)] = zero16
        r0 = sid * ROWS_PER_TILE
        for k in range(9):
            pltpu.sync_copy(rows_v, acc_sh.at[pl.ds(r0 + k * B, B)])
        rem = ROWS_PER_TILE - 9 * B
        pltpu.sync_copy(rows_v.at[pl.ds(0, rem)],
                        acc_sh.at[pl.ds(r0 + 9 * B, rem)])

        @pl.when(sid < 10)
        def _zero_deg():
            pltpu.sync_copy(zd_v, deg_sh.at[pl.ds(sid * DEG_CHUNK, DEG_CHUNK)])

        plsc.subcore_barrier()

        tbase = sid * T

        def batch_body(b, carry):
            off = b * B
            ws = []
            for g in range(B // L):
                sl = pl.ds(off + g * L, L)
                s16 = src_v[sl]
                d16 = dst_v[sl]
                a16 = attr_v[sl]
                gl = pl.ds(g * L, L)
                ridx_v[gl] = s16 + cid * N
                sidx_v[gl] = d16 + jnp.where(a16 < 0.0, N, 0)
                w16 = jnp.abs(a16)
                w_v[gl] = w16
                ws.append(w16)
            pltpu.async_copy(x2_hbm.at[ridx_v], rows_v, sem).wait()
            for g in range(B // L):
                w16 = ws[g]
                for i in range(L):
                    wv = w16.at[jnp.full((L,), i, jnp.int32)].get(
                        mode="promise_in_bounds")
                    e = g * L + i
                    for j in range(HALF // L):
                        cs = pl.ds(j * L, L)
                        rows_v[e, cs] = rows_v[e, cs] * wv
            pltpu.sync_copy(rows_v, acc_sh.at[sidx_v], add=True)
            pltpu.sync_copy(w_v, deg_sh.at[sidx_v], add=True)
            return carry

        def chunk_body(c, carry):
            cbase = tbase + c * CK
            pltpu.sync_copy(src_hbm.at[pl.ds(cbase, CK)], src_v)
            pltpu.sync_copy(dst_hbm.at[pl.ds(cbase, CK)], dst_v)
            pltpu.sync_copy(attr_hbm.at[pl.ds(cbase, CK)], attr_v)
            lax.fori_loop(0, CK // B, batch_body, 0)
            return carry

        lax.fori_loop(0, NCHUNK, chunk_body, 0)
        plsc.subcore_barrier()

        pltpu.sync_copy(acc_sh.at[pl.ds(r0, ROWS_PER_TILE)],
                        acc_out.at[cid, sid])

        @pl.when(sid < 10)
        def _write_deg():
            d0 = sid * DEG_CHUNK
            pltpu.sync_copy(deg_sh.at[pl.ds(d0, DEG_CHUNK)],
                            deg_out.at[cid, sid])

    return sc_kernel(x2, src, dst, attr)


def _tc_dense(acc, deg, x, W_pos_l, W_pos_r, b_pos, W_neg_l, W_neg_r, b_neg):
    R = 1000
    grid = (N // R,)

    def body(a_ref, deg_ref, x_ref, wpl, wpr, bp, wnl, wnr, bn, o_ref):
        a = a_ref[...]
        pos = jnp.concatenate([a[0, 0], a[1, 0]], axis=-1)
        neg = jnp.concatenate([a[0, 1], a[1, 1]], axis=-1)
        dg = deg_ref[0]
        dp = jnp.where(dg[0] > 0.0, dg[0], 1.0)
        dn = jnp.where(dg[1] > 0.0, dg[1], 1.0)
        pos = pos / dp[:, None]
        neg = neg / dn[:, None]
        xb = x_ref[...]
        dims = (((1,), (1,)), ((), ()))
        op = (lax.dot_general(pos, wpl[...], dims, preferred_element_type=jnp.float32)
              + lax.dot_general(xb, wpr[...], dims, preferred_element_type=jnp.float32)
              + bp[...])
        on = (lax.dot_general(neg, wnl[...], dims, preferred_element_type=jnp.float32)
              + lax.dot_general(xb, wnr[...], dims, preferred_element_type=jnp.float32)
              + bn[...])
        o_ref[...] = jnp.maximum(jnp.concatenate([op, on], axis=-1), 0.0)

    return pl.pallas_call(
        body,
        grid=grid,
        in_specs=[
            pl.BlockSpec((2, 2, R, HALF), lambda i: (0, 0, i, 0)),
            pl.BlockSpec((1, 2, R), lambda i: (i, 0, 0)),
            pl.BlockSpec((R, CH), lambda i: (i, 0)),
            pl.BlockSpec((CH, CH), lambda i: (0, 0)),
            pl.BlockSpec((CH, CH), lambda i: (0, 0)),
            pl.BlockSpec((1, CH), lambda i: (0, 0)),
            pl.BlockSpec((CH, CH), lambda i: (0, 0)),
            pl.BlockSpec((CH, CH), lambda i: (0, 0)),
            pl.BlockSpec((1, CH), lambda i: (0, 0)),
        ],
        out_specs=pl.BlockSpec((R, 2 * CH), lambda i: (i, 0)),
        out_shape=jax.ShapeDtypeStruct((N, 2 * CH), jnp.float32),
    )(acc, deg, x, W_pos_l, W_pos_r, b_pos.reshape(1, CH),
      W_neg_l, W_neg_r, b_neg.reshape(1, CH))


def kernel(x, edge_index, edge_attr, W_pos_l, W_pos_r, b_pos,
           W_neg_l, W_neg_r, b_neg):
    src = edge_index[0].astype(jnp.int32)
    dst = edge_index[1].astype(jnp.int32)
    pad = E_PAD - E
    src = jnp.pad(src, (0, pad))
    dst = jnp.pad(dst, (0, pad))
    attr = jnp.pad(edge_attr, (0, pad))
    x2 = jnp.concatenate([x[:, :HALF], x[:, HALF:]], axis=0)

    acc, deg = _sc_aggregate(x2, src, dst, attr)
    acc = acc.reshape(NC, 2, N, HALF)
    deg = deg[0].reshape(2, 10, 1000).transpose(1, 0, 2)

    return _tc_dense(acc, deg, x, W_pos_l, W_pos_r, b_pos,
                     W_neg_l, W_neg_r, b_neg)
